# baseline (speedup 1.0000x reference)
import functools

import jax
import jax.numpy as jnp
from jax import lax
from jax.experimental import pallas as pl
from jax.experimental.pallas import tpu as pltpu
from jax.experimental.pallas import tpu_sc as plsc

_SAMPLE_NUM = 1152
_EPS = 1e-5

try:
    _info = plsc.get_sparse_core_info()
    _NC, _NS = _info.num_cores, _info.num_subcores
except Exception:
    _NC, _NS = 2, 16
_NW = _NC * _NS


def _sc_gather_add(B, Ng, N, S, C, M):
    r0 = B * Ng // _NW
    r1 = B * S // _NW
    CH = 24
    n0 = r0 // CH
    n1 = r1 // CH
    WB = _NW // B
    mesh = plsc.VectorSubcoreMesh(core_axis_name="c", subcore_axis_name="s")

    def add_rows(xbuf, pfbuf):
        def add_row(r, carry):
            for c in range(0, C, 16):
                xbuf[r, pl.ds(c, 16)] = (xbuf[r, pl.ds(c, 16)]
                                         + pfbuf[r, pl.ds(c, 16)])
            return carry
        lax.fori_loop(0, CH, add_row, 0)

    @functools.partial(
        pl.kernel,
        mesh=mesh,
        out_type=jax.ShapeDtypeStruct((B * Ng, C), jnp.float32),
        scratch_types=[
            pltpu.VMEM((r0,), jnp.int32),
            pltpu.VMEM((CH, C), jnp.float32),
            pltpu.VMEM((CH, C), jnp.float32),
            pltpu.VMEM((CH, C), jnp.float32),
            pltpu.VMEM((CH, C), jnp.float32),
        ] + [pltpu.SemaphoreType.DMA] * 3,
        name="sc_grid_add",
    )
    def k0(xg_hbm, pg_hbm, pe_hbm, out0_hbm,
           pg_v, gx0, gpf0, gx1, gpf1, semA0, semA1, semG):
        wid = lax.axis_index("s") * _NC + lax.axis_index("c")
        b0 = wid * r0
        pltpu.sync_copy(pg_hbm.at[pl.ds(b0, r0)], pg_v)
        bufs = ((gx0, gpf0, semA0), (gx1, gpf1, semA1))
        pend = [None, None]
        stores = []

        def start0(kc):
            xb, pb, sem = bufs[kc % 2]
            cg = pltpu.async_copy(xg_hbm.at[pl.ds(b0 + kc * CH, CH)],
                                  xb, sem)
            cf = pltpu.async_copy(
                pe_hbm.at[pg_v.at[pl.ds(kc * CH, CH)]], pb, sem)
            pend[kc % 2] = (cg, cf)

        start0(0)
        start0(1)
        for kc in range(n0):
            xb, pb, _ = bufs[kc % 2]
            cg, cf = pend[kc % 2]
            cg.wait()
            cf.wait()
            add_rows(xb, pb)
            st = pltpu.async_copy(
                xb, out0_hbm.at[pl.ds(b0 + kc * CH, CH)], semG)
            stores.append(st)
            if kc + 2 < n0:
                st.wait()
                start0(kc + 2)
        for st in stores[-2:]:
            st.wait()

    @functools.partial(
        pl.kernel,
        mesh=mesh,
        out_type=jax.ShapeDtypeStruct((B * S, C), jnp.float32),
        scratch_types=[
            pltpu.VMEM((r1,), jnp.int32),
            pltpu.VMEM((r1,), jnp.int32),
            pltpu.VMEM((CH, C), jnp.float32),
            pltpu.VMEM((CH, C), jnp.float32),
            pltpu.VMEM((CH, C), jnp.float32),
            pltpu.VMEM((CH, C), jnp.float32),
        ] + [pltpu.SemaphoreType.DMA] * 4,
        name="sc_down_gather",
    )
    def k1(xa_hbm, idx_hbm, pos_hbm, pe_hbm, out1_hbm,
           idx_v, posd_v, ax0, apf0, ax1, apf1,
           semC, semE0, semE1, semG):
        wid = lax.axis_index("s") * _NC + lax.axis_index("c")
        boff = (wid // WB) * M + (wid % WB) * r1
        obase = wid * r1

        pltpu.sync_copy(idx_hbm.at[pl.ds(boff, r1)], idx_v)
        pltpu.async_copy(pos_hbm.at[idx_v], posd_v, semC).wait()

        abufs = ((ax0, apf0, semE0), (ax1, apf1, semE1))
        pend = [None, None]
        stores = []

        def start1(kc):
            xb, pb, sem = abufs[kc % 2]
            cx = pltpu.async_copy(
                xa_hbm.at[idx_v.at[pl.ds(kc * CH, CH)]], xb, sem)
            cp = pltpu.async_copy(
                pe_hbm.at[posd_v.at[pl.ds(kc * CH, CH)]], pb, sem)
            pend[kc % 2] = (cx, cp)

        start1(0)
        start1(1)
        for kc in range(n1):
            xb, pb, _ = abufs[kc % 2]
            cx, cp = pend[kc % 2]
            cx.wait()
            cp.wait()
            add_rows(xb, pb)
            st = pltpu.async_copy(
                xb, out1_hbm.at[pl.ds(obase + kc * CH, CH)], semG)
            stores.append(st)
            if kc + 2 < n1:
                st.wait()
                start1(kc + 2)
        for st in stores[-2:]:
            st.wait()

    return k0, k1


def _topk_sort_kernel(B, M):
    R = M // 128
    L = M.bit_length() - 1

    def body(key_ref, idx_out_ref):
        v = jnp.concatenate(
            [key_ref[...],
             jnp.full((B, R - key_ref.shape[1], 128), -1.0, jnp.float32)],
            axis=1)
        lane = lax.broadcasted_iota(jnp.int32, (B, R, 128), 2)
        rowi = lax.broadcasted_iota(jnp.int32, (B, R, 128), 1)
        bi = lax.broadcasted_iota(jnp.int32, (B, R, 128), 0)
        ix = bi * 2304 + rowi * 128 + lane

        for k in range(1, L + 1):
            size = 1 << k
            if size < 128:
                desc = (lane & size) == 0
            elif size < M:
                desc = (rowi & (size // 128)) == 0
            else:
                desc = jnp.full((B, R, 128), True)
            for j in range(k - 1, -1, -1):
                d = 1 << j
                if d < 128:
                    left = (lane & d) == 0
                    pv = jnp.where(left, jnp.roll(v, -d, axis=2),
                                   jnp.roll(v, d, axis=2))
                    pi = jnp.where(left, jnp.roll(ix, -d, axis=2),
                                   jnp.roll(ix, d, axis=2))
                else:
                    dr = d // 128
                    left = (rowi & dr) == 0
                    pv = jnp.where(left, jnp.roll(v, -dr, axis=1),
                                   jnp.roll(v, dr, axis=1))
                    pi = jnp.where(left, jnp.roll(ix, -dr, axis=1),
                                   jnp.roll(ix, dr, axis=1))
                win = (v > pv) | ((v == pv) & (ix < pi))
                m = (left == desc) == win
                v = jnp.where(m, v, pv)
                ix = jnp.where(m, ix, pi)
        idx_out_ref[...] = ix

    return pl.pallas_call(
        body,
        out_shape=jax.ShapeDtypeStruct((B, R, 128), jnp.int32),
    )


def kernel(x_grid, x_ada, pos_grid, pos_ada, pos_embed, norm_w, norm_b,
           conf_w, conf_b):
    B, N_g, C = x_grid.shape
    N = x_ada.shape[1]
    S = _SAMPLE_NUM

    mu = jnp.mean(x_ada, axis=-1, keepdims=True)
    var = jnp.var(x_ada, axis=-1, keepdims=True)
    normed = (x_ada - mu) / jnp.sqrt(var + _EPS) * norm_w + norm_b
    conf = normed @ conf_w + conf_b
    conf = jax.nn.softmax(conf, axis=1) * N

    s = conf[..., 0]
    M = 4096
    pg_f = pos_grid.reshape(-1).astype(jnp.int32)
    pe = pos_embed[0]

    k0, k1 = _sc_gather_add(B, N_g, N, S, C, M)
    out0_f = k0(x_grid.reshape(B * N_g, C), pg_f, pe)

    ix_sorted = _topk_sort_kernel(B, M)(s.reshape(B, N // 128, 128))
    out1_f = k1(x_ada.reshape(B * N, C), ix_sorted.reshape(B * M),
                pos_ada.reshape(-1).astype(jnp.int32), pe)
    out0 = out0_f.reshape(B, N_g, C)
    out1 = out1_f.reshape(B, S, C)
    return out0, out1, pos_grid, pos_ada


# speedup vs baseline: 1.0478x; 1.0478x over previous
import functools

import jax
import jax.numpy as jnp
from jax import lax
from jax.experimental import pallas as pl
from jax.experimental.pallas import tpu as pltpu
from jax.experimental.pallas import tpu_sc as plsc

_SAMPLE_NUM = 1152
_EPS = 1e-5

try:
    _info = plsc.get_sparse_core_info()
    _NC, _NS = _info.num_cores, _info.num_subcores
except Exception:
    _NC, _NS = 2, 16
_NW = _NC * _NS


def _sc_gather_add(B, Ng, N, S, C, M):
    r0 = B * Ng // _NW
    r1 = B * S // _NW
    CH = 24
    n0 = r0 // CH
    n1 = r1 // CH
    WB = _NW // B
    mesh = plsc.VectorSubcoreMesh(core_axis_name="c", subcore_axis_name="s")

    def add_rows(xbuf, pfbuf):
        def add_row(r, carry):
            for c in range(0, C, 16):
                xbuf[r, pl.ds(c, 16)] = (xbuf[r, pl.ds(c, 16)]
                                         + pfbuf[r, pl.ds(c, 16)])
            return carry
        lax.fori_loop(0, CH, add_row, 0)

    @functools.partial(
        pl.kernel,
        mesh=mesh,
        out_type=(
            jax.ShapeDtypeStruct((B * Ng, C), jnp.float32),
            jax.ShapeDtypeStruct((B * S, C), jnp.float32),
        ),
        scratch_types=[
            pltpu.VMEM((r0,), jnp.int32),
            pltpu.VMEM((r1,), jnp.int32),
            pltpu.VMEM((r1,), jnp.int32),
            pltpu.VMEM((CH, C), jnp.float32),
            pltpu.VMEM((CH, C), jnp.float32),
            pltpu.VMEM((CH, C), jnp.float32),
            pltpu.VMEM((CH, C), jnp.float32),
            pltpu.VMEM((CH, C), jnp.float32),
            pltpu.VMEM((CH, C), jnp.float32),
        ] + [pltpu.SemaphoreType.DMA] * 5,
        name="sc_gather_add",
    )
    def k(xg_hbm, pg_hbm, xa_hbm, idx_hbm, pos_hbm, pe_hbm,
          out0_hbm, out1_hbm,
          pg_v, idx_v, posd_v, ax0, apf0, ax1, apf1, gx0, gpf0,
          semC, semE0, semE1, semA0, semG):
        wid = lax.axis_index("s") * _NC + lax.axis_index("c")
        b0 = wid * r0
        boff = (wid // WB) * M + (wid % WB) * r1
        obase = wid * r1

        pltpu.sync_copy(idx_hbm.at[pl.ds(boff, r1)], idx_v)
        c_pd = pltpu.async_copy(pos_hbm.at[idx_v], posd_v, semC)
        pltpu.sync_copy(pg_hbm.at[pl.ds(b0, r0)], pg_v)
        c_pd.wait()

        abufs = ((ax0, apf0, semE0), (ax1, apf1, semE1))
        pend = [None, None]
        stores = []

        def start1(kc):
            xb, pb, sem = abufs[kc % 2]
            cx = pltpu.async_copy(
                xa_hbm.at[idx_v.at[pl.ds(kc * CH, CH)]], xb, sem)
            cp = pltpu.async_copy(
                pe_hbm.at[posd_v.at[pl.ds(kc * CH, CH)]], pb, sem)
            pend[kc % 2] = (cx, cp)

        start1(0)
        start1(1)
        for kc in range(n1):
            xb, pb, _ = abufs[kc % 2]
            cx, cp = pend[kc % 2]
            cx.wait()
            cp.wait()
            add_rows(xb, pb)
            st = pltpu.async_copy(
                xb, out1_hbm.at[pl.ds(obase + kc * CH, CH)], semG)
            stores.append(st)
            if kc + 2 < n1:
                st.wait()
                start1(kc + 2)

        for kc in range(n0):
            base = b0 + kc * CH
            cg = pltpu.async_copy(xg_hbm.at[pl.ds(base, CH)], gx0, semA0)
            cf = pltpu.async_copy(
                pe_hbm.at[pg_v.at[pl.ds(kc * CH, CH)]], gpf0, semA0)
            cg.wait()
            cf.wait()
            add_rows(gx0, gpf0)
            pltpu.sync_copy(gx0, out0_hbm.at[pl.ds(base, CH)])

        for st in stores[-2:]:
            st.wait()

    return k


def _topk_sort_kernel(B, M):
    R = M // 128
    L = M.bit_length() - 1

    def body(key_ref, idx_out_ref):
        v = jnp.concatenate(
            [key_ref[...],
             jnp.full((B, R - key_ref.shape[1], 128), -1.0, jnp.float32)],
            axis=1)
        lane = lax.broadcasted_iota(jnp.int32, (B, R, 128), 2)
        rowi = lax.broadcasted_iota(jnp.int32, (B, R, 128), 1)
        bi = lax.broadcasted_iota(jnp.int32, (B, R, 128), 0)
        ix = bi * 2304 + rowi * 128 + lane

        for k in range(1, L + 1):
            size = 1 << k
            if size < 128:
                desc = (lane & size) == 0
            elif size < M:
                desc = (rowi & (size // 128)) == 0
            else:
                desc = jnp.full((B, R, 128), True)
            for j in range(k - 1, -1, -1):
                d = 1 << j
                if d < 128:
                    left = (lane & d) == 0
                    pv = jnp.where(left, jnp.roll(v, -d, axis=2),
                                   jnp.roll(v, d, axis=2))
                    pi = jnp.where(left, jnp.roll(ix, -d, axis=2),
                                   jnp.roll(ix, d, axis=2))
                else:
                    dr = d // 128
                    left = (rowi & dr) == 0
                    pv = jnp.where(left, jnp.roll(v, -dr, axis=1),
                                   jnp.roll(v, dr, axis=1))
                    pi = jnp.where(left, jnp.roll(ix, -dr, axis=1),
                                   jnp.roll(ix, dr, axis=1))
                win = (v > pv) | ((v == pv) & (ix < pi))
                m = (left == desc) == win
                v = jnp.where(m, v, pv)
                ix = jnp.where(m, ix, pi)
        idx_out_ref[...] = ix

    return pl.pallas_call(
        body,
        out_shape=jax.ShapeDtypeStruct((B, R, 128), jnp.int32),
    )


def kernel(x_grid, x_ada, pos_grid, pos_ada, pos_embed, norm_w, norm_b,
           conf_w, conf_b):
    B, N_g, C = x_grid.shape
    N = x_ada.shape[1]
    S = _SAMPLE_NUM

    mu = jnp.mean(x_ada, axis=-1, keepdims=True)
    var = jnp.var(x_ada, axis=-1, keepdims=True)
    normed = (x_ada - mu) / jnp.sqrt(var + _EPS) * norm_w + norm_b
    conf = normed @ conf_w + conf_b
    conf = jax.nn.softmax(conf, axis=1) * N

    s = conf[..., 0]
    M = 4096
    pg_f = pos_grid.reshape(-1).astype(jnp.int32)
    pe = pos_embed[0]

    ix_sorted = _topk_sort_kernel(B, M)(s.reshape(B, N // 128, 128))
    sc = _sc_gather_add(B, N_g, N, S, C, M)
    out0_f, out1_f = sc(
        x_grid.reshape(B * N_g, C), pg_f,
        x_ada.reshape(B * N, C), ix_sorted.reshape(B * M),
        pos_ada.reshape(-1).astype(jnp.int32), pe)
    out0 = out0_f.reshape(B, N_g, C)
    out1 = out1_f.reshape(B, S, C)
    return out0, out1, pos_grid, pos_ada
